# Initial kernel scaffold; baseline (speedup 1.0000x reference)
import jax
import jax.numpy as jnp
from jax.experimental import pallas as pl


def kernel(local_feat, attention_map):
    raise NotImplementedError("write your pallas kernel here")


# speedup vs baseline: 1.1639x; 1.1639x over previous
import jax
import jax.numpy as jnp
from jax.experimental import pallas as pl
from jax.experimental.pallas import tpu as pltpu


def _interp_matrix(in_size):
    out_size = 2 * in_size
    o = jnp.arange(out_size, dtype=jnp.float32)
    src = jnp.maximum((o + 0.5) / 2.0 - 0.5, 0.0)
    i0 = jnp.clip(jnp.floor(src).astype(jnp.int32), 0, in_size - 1)
    i1 = jnp.minimum(i0 + 1, in_size - 1)
    w1 = src - i0.astype(jnp.float32)
    w0 = 1.0 - w1
    i = jnp.arange(in_size, dtype=jnp.int32)[None, :]
    return w0[:, None] * (i == i0[:, None]) + w1[:, None] * (i == i1[:, None])


def _make_body(HT):
    def body(att_ref, r_ref, c_ref, lf_ref, out_ref, mask_ref, mscratch):
        h = pl.program_id(1)
        c = pl.program_id(2)

        @pl.when(jnp.logical_and(h == 0, c == 0))
        def _():
            x = att_ref[0, 0]
            up = jnp.dot(
                jnp.dot(r_ref[...], x, preferred_element_type=jnp.float32),
                c_ref[...],
                preferred_element_type=jnp.float32,
            )
            mscratch[...] = jax.nn.sigmoid(10.0 * (up - 0.5))

        mblk = mscratch[pl.ds(h * HT, HT), :]

        @pl.when(c == 0)
        def _():
            mask_ref[0, 0] = mblk

        out_ref[...] = lf_ref[...] * (mblk[None, None] + 0.1)

    return body


def kernel(local_feat, attention_map):
    B, C, H, W = local_feat.shape
    Hi = attention_map.shape[2]
    Wi = attention_map.shape[3]

    rmat = _interp_matrix(Hi)
    cmat = _interp_matrix(Wi).T

    CT = 48
    HT = 192
    out, mask = pl.pallas_call(
        _make_body(HT),
        grid=(B, H // HT, C // CT),
        in_specs=[
            pl.BlockSpec((1, 1, Hi, Wi), lambda b, h, c: (b, 0, 0, 0)),
            pl.BlockSpec((H, Hi), lambda b, h, c: (0, 0)),
            pl.BlockSpec((Wi, W), lambda b, h, c: (0, 0)),
            pl.BlockSpec((1, CT, HT, W), lambda b, h, c: (b, c, h, 0)),
        ],
        out_specs=[
            pl.BlockSpec((1, CT, HT, W), lambda b, h, c: (b, c, h, 0)),
            pl.BlockSpec((1, 1, HT, W), lambda b, h, c: (b, 0, h, 0)),
        ],
        out_shape=[
            jax.ShapeDtypeStruct((B, C, H, W), jnp.float32),
            jax.ShapeDtypeStruct((B, 1, H, W), jnp.float32),
        ],
        scratch_shapes=[pltpu.VMEM((H, W), jnp.float32)],
    )(attention_map, rmat, cmat, local_feat)

    return (out, mask)
